# baseline (speedup 1.0000x reference)
import jax
import jax.numpy as jnp
from jax.experimental import pallas as pl

_MIN_DIST = 8.0
_STEP_DIST = 0.1
_NUM_BINS = 140
_BL = 8


def _tc_body(x_ref, o_ref):
    x = x_ref[...]
    idx = ((x - _MIN_DIST) / _STEP_DIST).astype(jnp.int32)
    idx = jnp.minimum(jnp.maximum(idx, 0), _NUM_BINS - 1)
    k = jax.lax.broadcasted_iota(jnp.int32, (1, _BL, 256, _NUM_BINS), 3)
    o_ref[...] = (idx[..., None] == k).astype(jnp.float32)


def kernel(ipt, table):
    del table
    return pl.pallas_call(
        _tc_body,
        grid=(8, 256 // _BL),
        in_specs=[pl.BlockSpec((1, _BL, 256), lambda b, h: (b, h, 0))],
        out_specs=pl.BlockSpec((1, _BL, 256, _NUM_BINS),
                               lambda b, h: (b, h, 0, 0)),
        out_shape=jax.ShapeDtypeStruct((8, 256, 256, _NUM_BINS), jnp.float32),
    )(ipt)


# speedup vs baseline: 1.1208x; 1.1208x over previous
import jax
import jax.numpy as jnp
from jax.experimental import pallas as pl

_MIN_DIST = 8.0
_STEP_DIST = 0.1
_NUM_BINS = 140
_BL = 64


def _tc_body(x_ref, o_ref):
    x = x_ref[...]
    idx = ((x - _MIN_DIST) / _STEP_DIST).astype(jnp.int32)
    idx = jnp.minimum(jnp.maximum(idx, 0), _NUM_BINS - 1)
    k = jax.lax.broadcasted_iota(jnp.int32, (1, _BL, 256, _NUM_BINS), 3)
    o_ref[...] = (idx[..., None] == k).astype(jnp.float32)


def kernel(ipt, table):
    del table
    return pl.pallas_call(
        _tc_body,
        grid=(8, 256 // _BL),
        in_specs=[pl.BlockSpec((1, _BL, 256), lambda b, h: (b, h, 0))],
        out_specs=pl.BlockSpec((1, _BL, 256, _NUM_BINS),
                               lambda b, h: (b, h, 0, 0)),
        out_shape=jax.ShapeDtypeStruct((8, 256, 256, _NUM_BINS), jnp.float32),
    )(ipt)
